# baseline (speedup 1.0000x reference)
import jax
import jax.numpy as jnp
from jax import lax
from jax.experimental import pallas as pl
from jax.experimental.pallas import tpu as pltpu

B, F = 2048, 512
OUT, OR_T, AND_T = 1024, 8, 4
C = OUT * OR_T


def _fused_kernel(wk_ref, mask_ref, x_ref, o_ref, d_s, e_s):
    @pl.when(pl.program_id(0) == 0)
    def _prep():
        iota = lax.broadcasted_iota(jnp.int32, (F, C), 0)
        d = jnp.zeros((F, C), jnp.int32)
        e = jnp.zeros((1, C), jnp.int32)
        for k in range(AND_T):
            wk = wk_ref[k : k + 1, :]
            q = (wk - 1) & (F - 1)
            sgn_i = (wk >= 1).astype(jnp.int32) * (1 - 2 * (wk > F).astype(jnp.int32))
            d = d + (q == iota).astype(jnp.int32) * sgn_i
            e = e + (wk == 0).astype(jnp.int32) + (wk > F).astype(jnp.int32)
        d_s[...] = d.astype(jnp.int8)
        thr = jnp.where(mask_ref[...] != 0, 4 - e, 1000000)
        e_s[...] = thr

    xb = (x_ref[...] != 0.0).astype(jnp.int8)
    s = jnp.dot(xb, d_s[...], preferred_element_type=jnp.int32)
    acc = s[:, 0:OUT] >= e_s[0:1, 0:OUT]
    for t in range(1, OR_T):
        acc = acc | (s[:, t * OUT : (t + 1) * OUT] >= e_s[0:1, t * OUT : (t + 1) * OUT])
    o_ref[...] = acc


@jax.jit
def kernel(x, weights, or_padding_mask):
    wk = weights.transpose(2, 1, 0).reshape(AND_T, C)
    mask = or_padding_mask.transpose(1, 0).reshape(1, C).astype(jnp.int32)

    bb = 512
    out = pl.pallas_call(
        _fused_kernel,
        grid=(B // bb,),
        in_specs=[
            pl.BlockSpec((AND_T, C), lambda i: (0, 0)),
            pl.BlockSpec((1, C), lambda i: (0, 0)),
            pl.BlockSpec((bb, F), lambda i: (i, 0)),
        ],
        out_specs=pl.BlockSpec((bb, OUT), lambda i: (i, 0)),
        out_shape=jax.ShapeDtypeStruct((B, OUT), jnp.bool_),
        scratch_shapes=[
            pltpu.VMEM((F, C), jnp.int8),
            pltpu.VMEM((1, C), jnp.int32),
        ],
    )(wk, mask, x)

    return out


# speedup vs baseline: 1.1737x; 1.1737x over previous
import jax
import jax.numpy as jnp
from jax import lax
from jax.experimental import pallas as pl
from jax.experimental.pallas import tpu as pltpu

B, F = 2048, 512
OUT, OR_T, AND_T = 1024, 8, 4
C = OUT * OR_T


def _fused_kernel(wk_ref, mask_ref, x_ref, o_ref, d_s, e_s):
    xb = (x_ref[...] != 0.0).astype(jnp.bfloat16)

    @pl.when(pl.program_id(0) == 0)
    def _first():
        iota = lax.broadcasted_iota(jnp.int32, (F, OUT), 0)
        acc = None
        for t in range(OR_T):
            lo, hi = t * OUT, (t + 1) * OUT
            d = jnp.zeros((F, OUT), jnp.bfloat16)
            e = jnp.zeros((1, OUT), jnp.float32)
            for k in range(AND_T):
                wk = wk_ref[k : k + 1, lo:hi]
                q = (wk - 1) & (F - 1)
                sgn_i = (wk >= 1).astype(jnp.int32) * (1 - 2 * (wk > F).astype(jnp.int32))
                d = d + (q == iota).astype(jnp.bfloat16) * sgn_i.astype(jnp.bfloat16)
                e = e + (wk == 0).astype(jnp.float32) + (wk > F).astype(jnp.float32)
            thr = jnp.where(mask_ref[0:1, lo:hi] != 0, 3.5 - e, 100000.0)
            d_s[:, lo:hi] = d
            e_s[0:1, lo:hi] = thr.astype(jnp.bfloat16)
            s_t = jnp.dot(xb, d, preferred_element_type=jnp.float32)
            a_t = s_t >= thr
            acc = a_t if acc is None else acc | a_t
        o_ref[...] = acc

    @pl.when(pl.program_id(0) > 0)
    def _rest():
        s = jnp.dot(xb, d_s[...], preferred_element_type=jnp.float32)
        acc = s[:, 0:OUT] >= e_s[0:1, 0:OUT]
        for t in range(1, OR_T):
            acc = acc | (s[:, t * OUT : (t + 1) * OUT] >= e_s[0:1, t * OUT : (t + 1) * OUT])
        o_ref[...] = acc


@jax.jit
def kernel(x, weights, or_padding_mask):
    wk = weights.transpose(2, 1, 0).reshape(AND_T, C)
    mask = or_padding_mask.transpose(1, 0).reshape(1, C).astype(jnp.int32)

    bb = 512
    out = pl.pallas_call(
        _fused_kernel,
        grid=(B // bb,),
        in_specs=[
            pl.BlockSpec((AND_T, C), lambda i: (0, 0)),
            pl.BlockSpec((1, C), lambda i: (0, 0)),
            pl.BlockSpec((bb, F), lambda i: (i, 0)),
        ],
        out_specs=pl.BlockSpec((bb, OUT), lambda i: (i, 0)),
        out_shape=jax.ShapeDtypeStruct((B, OUT), jnp.bool_),
        scratch_shapes=[
            pltpu.VMEM((F, C), jnp.bfloat16),
            pltpu.VMEM((1, C), jnp.bfloat16),
        ],
    )(wk, mask, x)

    return out


# speedup vs baseline: 1.3546x; 1.1541x over previous
import jax
import jax.numpy as jnp
from jax import lax
from jax.experimental import pallas as pl
from jax.experimental.pallas import tpu as pltpu

B, F = 2048, 512
OUT, OR_T, AND_T = 1024, 8, 4
C = OUT * OR_T


def _fused_kernel(wk_ref, mask_ref, x_ref, o_ref, d_s, e_s):
    xb = (x_ref[...] != 0.0).astype(jnp.bfloat16)

    @pl.when(pl.program_id(0) == 0)
    def _first():
        iota = lax.broadcasted_iota(jnp.int16, (F, OUT), 0)
        acc = None
        for t in range(OR_T):
            lo, hi = t * OUT, (t + 1) * OUT
            d = jnp.zeros((F, OUT), jnp.bfloat16)
            e = jnp.zeros((1, OUT), jnp.float32)
            for k in range(AND_T):
                wk = wk_ref[k : k + 1, lo:hi]
                q = ((wk - 1) & (F - 1)).astype(jnp.int16)
                sgn_i = (wk >= 1).astype(jnp.int32) * (1 - 2 * (wk > F).astype(jnp.int32))
                sgn_b = jnp.broadcast_to(sgn_i.astype(jnp.bfloat16), (F, OUT))
                d = jnp.where(q == iota, sgn_b + d, d)
                e = e + (wk == 0).astype(jnp.float32) + (wk > F).astype(jnp.float32)
            thr = jnp.where(mask_ref[0:1, lo:hi] != 0, 3.5 - e, 100000.0)
            d_s[:, lo:hi] = d
            e_s[0:1, lo:hi] = thr.astype(jnp.bfloat16)
            s_t = jnp.dot(xb, d, preferred_element_type=jnp.float32)
            a_t = s_t >= thr
            acc = a_t if acc is None else acc | a_t
        o_ref[...] = acc

    @pl.when(pl.program_id(0) > 0)
    def _rest():
        s = jnp.dot(xb, d_s[...], preferred_element_type=jnp.float32)
        acc = s[:, 0:OUT] >= e_s[0:1, 0:OUT]
        for t in range(1, OR_T):
            acc = acc | (s[:, t * OUT : (t + 1) * OUT] >= e_s[0:1, t * OUT : (t + 1) * OUT])
        o_ref[...] = acc


@jax.jit
def kernel(x, weights, or_padding_mask):
    wk = weights.transpose(2, 1, 0).reshape(AND_T, C)
    mask = or_padding_mask.transpose(1, 0).reshape(1, C).astype(jnp.int32)

    bb = 512
    out = pl.pallas_call(
        _fused_kernel,
        grid=(B // bb,),
        in_specs=[
            pl.BlockSpec((AND_T, C), lambda i: (0, 0)),
            pl.BlockSpec((1, C), lambda i: (0, 0)),
            pl.BlockSpec((bb, F), lambda i: (i, 0)),
        ],
        out_specs=pl.BlockSpec((bb, OUT), lambda i: (i, 0)),
        out_shape=jax.ShapeDtypeStruct((B, OUT), jnp.bool_),
        scratch_shapes=[
            pltpu.VMEM((F, C), jnp.bfloat16),
            pltpu.VMEM((1, C), jnp.bfloat16),
        ],
    )(wk, mask, x)

    return out
